# baseline (speedup 1.0000x reference)
import functools

import jax
import jax.numpy as jnp
from jax import lax
from jax.experimental import pallas as pl
from jax.experimental.pallas import tpu as pltpu
from jax.experimental.pallas import tpu_sc as plsc

N = 10000
E = 320000
F_IN = 128
H = 64
C = 10
G = 16

NC = 2
NS = 16
NW = NC * NS
ET = E // NW
K = 128
EP = 327680
ETP = EP // NW
STEPS = ETP // K
NP = 10240
RPT = NP // NS

_mesh = plsc.VectorSubcoreMesh(core_axis_name="c", subcore_axis_name="s")


def _deg_body(dst_hbm, zeros_hbm, out_hbm, idx_v, hist_v):
    cid = lax.axis_index("c")
    sid = lax.axis_index("s")
    wid = cid * NS + sid
    pltpu.sync_copy(zeros_hbm, hist_v)
    pltpu.sync_copy(dst_hbm.at[pl.ds(pl.multiple_of(wid * ET, 8), ET)], idx_v)
    ones16 = jnp.ones((16,), jnp.float32)

    def step(i, carry):
        idx16 = idx_v[pl.ds(i * 16, 16)]
        plsc.addupdate_scatter(hist_v, [idx16], ones16)
        return carry

    lax.fori_loop(0, ET // 16, step, 0)
    pltpu.sync_copy(hist_v, out_hbm.at[pl.ds(pl.multiple_of(wid * N, 8), N)])


_deg_call = pl.kernel(
    _deg_body,
    out_type=jax.ShapeDtypeStruct((NW * N,), jnp.float32),
    mesh=_mesh,
    scratch_types=[
        pltpu.VMEM((ET,), jnp.int32),
        pltpu.VMEM((N,), jnp.float32),
    ],
    compiler_params=pltpu.CompilerParams(needs_layout_passes=False),
)


HP = 128


def _scat_body(u_hbm, src_hbm, dst_hbm, zeros_hbm, out_hbm,
               six0, six1, dix0, dix1, rows0, rows1, acc_sh,
               sem_g0, sem_g1):
    cid = lax.axis_index("c")
    sid = lax.axis_index("s")
    wid = cid * NS + sid
    pltpu.sync_copy(zeros_hbm.at[pl.ds(pl.multiple_of(sid * RPT, 8), RPT)],
                    acc_sh.at[pl.ds(pl.multiple_of(sid * RPT, 8), RPT)])
    plsc.subcore_barrier()
    base = pl.multiple_of(wid * ETP, 8)

    def step(i, carry):
        off = pl.multiple_of(base + i * K, 8)
        pltpu.sync_copy(src_hbm.at[pl.ds(off, K)], six0)
        pltpu.sync_copy(dst_hbm.at[pl.ds(off, K)], dix0)
        pltpu.async_copy(u_hbm.at[six0], rows0, sem_g0).wait()
        pltpu.sync_copy(rows0, acc_sh.at[dix0], add=True)
        return carry

    lax.fori_loop(0, STEPS, step, 0)
    plsc.subcore_barrier()
    pltpu.sync_copy(acc_sh.at[pl.ds(pl.multiple_of(sid * RPT, 8), RPT)],
                    out_hbm.at[pl.ds(pl.multiple_of(cid * NP + sid * RPT, 8), RPT)])


_scat_call = pl.kernel(
    _scat_body,
    out_type=jax.ShapeDtypeStruct((2 * NP, HP), jnp.float32),
    mesh=_mesh,
    scratch_types=[
        pltpu.VMEM((K,), jnp.int32),
        pltpu.VMEM((K,), jnp.int32),
        pltpu.VMEM((K,), jnp.int32),
        pltpu.VMEM((K,), jnp.int32),
        pltpu.VMEM((K, HP), jnp.float32),
        pltpu.VMEM((K, HP), jnp.float32),
        pltpu.VMEM_SHARED((NP, HP), jnp.float32),
        pltpu.SemaphoreType.DMA,
        pltpu.SemaphoreType.DMA,
    ],
)


def _tc_prep_body(degp_ref, x_ref, w1_ref, u1_ref, d_ref):
    deg = jnp.sum(degp_ref[...], axis=1, keepdims=True) + 1.0
    d = lax.rsqrt(deg)
    h = jnp.dot(x_ref[...], w1_ref[...], preferred_element_type=jnp.float32)
    u1_ref[:, 0:H] = h * d
    u1_ref[:, H:HP] = jnp.zeros((N, HP - H), jnp.float32)
    d_ref[...] = d


def _tc_mid_body(sp_ref, u_ref, d_ref, b_ref, w2_ref, u2_ref):
    d = d_ref[...]
    s = (sp_ref[0:N, 0:H] + sp_ref[NP:NP + N, 0:H] + u_ref[0:N, 0:H])
    h = jnp.maximum(d * s + b_ref[...], 0.0)
    u2_ref[:, 0:H] = jnp.dot(h, w2_ref[...],
                             preferred_element_type=jnp.float32) * d
    u2_ref[:, H:HP] = jnp.zeros((N, HP - H), jnp.float32)


def _tc_final_body(sp_ref, u_ref, d_ref, b_ref, batch_ref, wl_ref, bl_ref,
                   out_ref):
    d = d_ref[...]
    s = (sp_ref[0:N, 0:H] + sp_ref[NP:NP + N, 0:H] + u_ref[0:N, 0:H])
    h = jnp.maximum(d * s + b_ref[...], 0.0)
    gids = lax.broadcasted_iota(jnp.int32, (1, G), 1)
    onehot = (batch_ref[...] == gids).astype(jnp.float32)
    sums = lax.dot_general(onehot, h, (((0,), (0,)), ((), ())),
                           preferred_element_type=jnp.float32)
    counts = jnp.sum(onehot, axis=0, keepdims=True)
    pooled = sums / jnp.maximum(counts, 1.0).reshape(G, 1)
    out_ref[...] = jnp.dot(pooled, wl_ref[...],
                           preferred_element_type=jnp.float32) + bl_ref[...]


def _tc_call(body, out_shape, n_in):
    return pl.pallas_call(
        body,
        out_shape=out_shape,
        in_specs=[pl.BlockSpec(memory_space=pltpu.VMEM)] * n_in,
        out_specs=(pl.BlockSpec(memory_space=pltpu.VMEM)
                   if not isinstance(out_shape, (list, tuple))
                   else [pl.BlockSpec(memory_space=pltpu.VMEM)] * len(out_shape)),
    )


_prep = _tc_call(_tc_prep_body,
                 [jax.ShapeDtypeStruct((N, HP), jnp.float32),
                  jax.ShapeDtypeStruct((N, 1), jnp.float32)], 3)
_mid = _tc_call(_tc_mid_body, jax.ShapeDtypeStruct((N, HP), jnp.float32), 5)
_final = _tc_call(_tc_final_body, jax.ShapeDtypeStruct((G, C), jnp.float32), 7)


@jax.jit
def kernel(x, edge_index, batch, W1, b1, W2, b2, Wl, bl):
    src = edge_index[0].astype(jnp.int32)
    dst = edge_index[1].astype(jnp.int32)
    npad = EP - E
    src_p = jnp.concatenate([src, jnp.zeros((npad,), jnp.int32)])
    dst_p = jnp.concatenate(
        [dst, N + (jnp.arange(npad, dtype=jnp.int32) % (NP - N))])

    zeros_nh = jnp.zeros((NP, HP), jnp.float32)
    zeros_n = jnp.zeros((N,), jnp.float32)

    deg_parts = _deg_call(dst, zeros_n).reshape(NW, N).T
    u1, d = _prep(deg_parts, x, W1)
    s1 = _scat_call(u1, src_p, dst_p, zeros_nh)
    u2 = _mid(s1, u1, d, b1.reshape(1, H), W2)
    s2 = _scat_call(u2, src_p, dst_p, zeros_nh)
    return _final(s2, u2, d, b2.reshape(1, H),
                  batch.astype(jnp.int32).reshape(N, 1), Wl,
                  bl.reshape(1, C))


# speedup vs baseline: 1.1647x; 1.1647x over previous
import functools

import jax
import jax.numpy as jnp
from jax import lax
from jax.experimental import pallas as pl
from jax.experimental.pallas import tpu as pltpu
from jax.experimental.pallas import tpu_sc as plsc

N = 10000
E = 320000
F_IN = 128
H = 64
C = 10
G = 16

NC = 2
NS = 16
NW = NC * NS
ET = E // NW
K = 80
EP = 327680
ETP = EP // NW
STEPS = ETP // K
NP = 10240
RPT = NP // NS

_mesh = plsc.VectorSubcoreMesh(core_axis_name="c", subcore_axis_name="s")


def _deg_body(dst_hbm, zeros_hbm, out_hbm, idx_v, hist_v):
    cid = lax.axis_index("c")
    sid = lax.axis_index("s")
    wid = cid * NS + sid
    pltpu.sync_copy(zeros_hbm, hist_v)
    pltpu.sync_copy(dst_hbm.at[pl.ds(pl.multiple_of(wid * ET, 8), ET)], idx_v)
    ones16 = jnp.ones((16,), jnp.float32)

    def step(i, carry):
        idx16 = idx_v[pl.ds(i * 16, 16)]
        plsc.addupdate_scatter(hist_v, [idx16], ones16)
        return carry

    lax.fori_loop(0, ET // 16, step, 0)
    pltpu.sync_copy(hist_v, out_hbm.at[pl.ds(pl.multiple_of(wid * N, 8), N)])


_deg_call = pl.kernel(
    _deg_body,
    out_type=jax.ShapeDtypeStruct((NW * N,), jnp.float32),
    mesh=_mesh,
    scratch_types=[
        pltpu.VMEM((ET,), jnp.int32),
        pltpu.VMEM((N,), jnp.float32),
    ],
    compiler_params=pltpu.CompilerParams(needs_layout_passes=False),
)


HP = 128


def _scat_body(u_hbm, src_hbm, dst_hbm, zeros_hbm, out_hbm,
               six0, six1, dix0, dix1, rows0, rows1, acc_sh,
               sem_g0, sem_g1):
    cid = lax.axis_index("c")
    sid = lax.axis_index("s")
    wid = cid * NS + sid
    pltpu.sync_copy(zeros_hbm.at[pl.ds(pl.multiple_of(sid * RPT, 8), RPT)],
                    acc_sh.at[pl.ds(pl.multiple_of(sid * RPT, 8), RPT)])
    plsc.subcore_barrier()

    six = (six0, six1)
    dix = (dix0, dix1)
    rows = (rows0, rows1)
    sem_g = (sem_g0, sem_g1)
    base = pl.multiple_of(wid * ETP, 8)

    pltpu.sync_copy(src_hbm.at[pl.ds(base, K)], six0)
    pltpu.sync_copy(dst_hbm.at[pl.ds(base, K)], dix0)
    pltpu.async_copy(u_hbm.at[six0], rows0, sem_g0)

    def pair(q, carry):
        for b in range(2):
            i = q * 2 + b
            ob = 1 - b

            @pl.when(i + 1 < STEPS)
            def _():
                off = pl.multiple_of(base + (i + 1) * K, 8)
                pltpu.sync_copy(src_hbm.at[pl.ds(off, K)], six[ob])
                pltpu.sync_copy(dst_hbm.at[pl.ds(off, K)], dix[ob])

            pltpu.make_async_copy(u_hbm.at[pl.ds(0, K)], rows[b],
                                  sem_g[b]).wait()

            @pl.when(i + 1 < STEPS)
            def _():
                pltpu.async_copy(u_hbm.at[six[ob]], rows[ob], sem_g[ob])

            pltpu.sync_copy(rows[b], acc_sh.at[dix[b]], add=True)
        return carry

    lax.fori_loop(0, STEPS // 2, pair, 0)
    plsc.subcore_barrier()
    pltpu.sync_copy(acc_sh.at[pl.ds(pl.multiple_of(sid * RPT, 8), RPT)],
                    out_hbm.at[pl.ds(pl.multiple_of(cid * NP + sid * RPT, 8), RPT)])


_scat_call = pl.kernel(
    _scat_body,
    out_type=jax.ShapeDtypeStruct((2 * NP, HP), jnp.float32),
    mesh=_mesh,
    scratch_types=[
        pltpu.VMEM((K,), jnp.int32),
        pltpu.VMEM((K,), jnp.int32),
        pltpu.VMEM((K,), jnp.int32),
        pltpu.VMEM((K,), jnp.int32),
        pltpu.VMEM((K, HP), jnp.float32),
        pltpu.VMEM((K, HP), jnp.float32),
        pltpu.VMEM_SHARED((NP, HP), jnp.float32),
        pltpu.SemaphoreType.DMA,
        pltpu.SemaphoreType.DMA,
    ],
)


def _tc_prep_body(degp_ref, x_ref, w1_ref, u1_ref, d_ref):
    deg = jnp.sum(degp_ref[...], axis=1, keepdims=True) + 1.0
    d = lax.rsqrt(deg)
    h = jnp.dot(x_ref[...], w1_ref[...], preferred_element_type=jnp.float32)
    u1_ref[:, 0:H] = h * d
    u1_ref[:, H:HP] = jnp.zeros((N, HP - H), jnp.float32)
    d_ref[...] = d


def _tc_mid_body(sp_ref, u_ref, d_ref, b_ref, w2_ref, u2_ref):
    d = d_ref[...]
    s = (sp_ref[0:N, 0:H] + sp_ref[NP:NP + N, 0:H] + u_ref[0:N, 0:H])
    h = jnp.maximum(d * s + b_ref[...], 0.0)
    u2_ref[:, 0:H] = jnp.dot(h, w2_ref[...],
                             preferred_element_type=jnp.float32) * d
    u2_ref[:, H:HP] = jnp.zeros((N, HP - H), jnp.float32)


def _tc_final_body(sp_ref, u_ref, d_ref, b_ref, batch_ref, wl_ref, bl_ref,
                   out_ref):
    d = d_ref[...]
    s = (sp_ref[0:N, 0:H] + sp_ref[NP:NP + N, 0:H] + u_ref[0:N, 0:H])
    h = jnp.maximum(d * s + b_ref[...], 0.0)
    gids = lax.broadcasted_iota(jnp.int32, (1, G), 1)
    onehot = (batch_ref[...] == gids).astype(jnp.float32)
    sums = lax.dot_general(onehot, h, (((0,), (0,)), ((), ())),
                           preferred_element_type=jnp.float32)
    counts = jnp.sum(onehot, axis=0, keepdims=True)
    pooled = sums / jnp.maximum(counts, 1.0).reshape(G, 1)
    out_ref[...] = jnp.dot(pooled, wl_ref[...],
                           preferred_element_type=jnp.float32) + bl_ref[...]


def _tc_call(body, out_shape, n_in):
    return pl.pallas_call(
        body,
        out_shape=out_shape,
        in_specs=[pl.BlockSpec(memory_space=pltpu.VMEM)] * n_in,
        out_specs=(pl.BlockSpec(memory_space=pltpu.VMEM)
                   if not isinstance(out_shape, (list, tuple))
                   else [pl.BlockSpec(memory_space=pltpu.VMEM)] * len(out_shape)),
    )


_prep = _tc_call(_tc_prep_body,
                 [jax.ShapeDtypeStruct((N, HP), jnp.float32),
                  jax.ShapeDtypeStruct((N, 1), jnp.float32)], 3)
_mid = _tc_call(_tc_mid_body, jax.ShapeDtypeStruct((N, HP), jnp.float32), 5)
_final = _tc_call(_tc_final_body, jax.ShapeDtypeStruct((G, C), jnp.float32), 7)


@jax.jit
def kernel(x, edge_index, batch, W1, b1, W2, b2, Wl, bl):
    src = edge_index[0].astype(jnp.int32)
    dst = edge_index[1].astype(jnp.int32)
    npad = EP - E
    src_p = jnp.concatenate([src, jnp.zeros((npad,), jnp.int32)])
    dst_p = jnp.concatenate(
        [dst, N + (jnp.arange(npad, dtype=jnp.int32) % (NP - N))])

    zeros_nh = jnp.zeros((NP, HP), jnp.float32)
    zeros_n = jnp.zeros((N,), jnp.float32)

    deg_parts = _deg_call(dst, zeros_n).reshape(NW, N).T
    u1, d = _prep(deg_parts, x, W1)
    s1 = _scat_call(u1, src_p, dst_p, zeros_nh)
    u2 = _mid(s1, u1, d, b1.reshape(1, H), W2)
    s2 = _scat_call(u2, src_p, dst_p, zeros_nh)
    return _final(s2, u2, d, b2.reshape(1, H),
                  batch.astype(jnp.int32).reshape(N, 1), Wl,
                  bl.reshape(1, C))


# speedup vs baseline: 1.6592x; 1.4246x over previous
import functools

import jax
import jax.numpy as jnp
from jax import lax
from jax.experimental import pallas as pl
from jax.experimental.pallas import tpu as pltpu
from jax.experimental.pallas import tpu_sc as plsc

N = 10000
E = 320000
F_IN = 128
H = 64
C = 10
G = 16

NC = 2
NS = 16
NW = NC * NS
ET = E // NW
K = 80
EP = 327680
ETP = EP // NW
STEPS = ETP // K
NP = 10240
RPT = NP // NS

_mesh = plsc.VectorSubcoreMesh(core_axis_name="c", subcore_axis_name="s")


def _deg_body(dst_hbm, zeros_hbm, out_hbm, idx_v, hist_v):
    cid = lax.axis_index("c")
    sid = lax.axis_index("s")
    wid = cid * NS + sid
    pltpu.sync_copy(zeros_hbm, hist_v)
    pltpu.sync_copy(dst_hbm.at[pl.ds(pl.multiple_of(wid * ET, 8), ET)], idx_v)
    ones16 = jnp.ones((16,), jnp.float32)

    def step(i, carry):
        idx16 = idx_v[pl.ds(i * 16, 16)]
        plsc.addupdate_scatter(hist_v, [idx16], ones16)
        return carry

    lax.fori_loop(0, ET // 16, step, 0)
    pltpu.sync_copy(hist_v, out_hbm.at[pl.ds(pl.multiple_of(wid * N, 8), N)])


_deg_call = pl.kernel(
    _deg_body,
    out_type=jax.ShapeDtypeStruct((NW * N,), jnp.float32),
    mesh=_mesh,
    scratch_types=[
        pltpu.VMEM((ET,), jnp.int32),
        pltpu.VMEM((N,), jnp.float32),
    ],
    compiler_params=pltpu.CompilerParams(needs_layout_passes=False),
)


HP = 128


def _scat_body(u_hbm, src_hbm, dst_hbm, zeros_hbm, out_hbm,
               six0, six1, dix0, dix1, rows0, rows1, acc_sh,
               sem_g0, sem_g1):
    cid = lax.axis_index("c")
    sid = lax.axis_index("s")
    wid = cid * NS + sid
    pltpu.sync_copy(zeros_hbm.at[pl.ds(pl.multiple_of(sid * RPT, 8), RPT)],
                    acc_sh.at[pl.ds(pl.multiple_of(sid * RPT, 8), RPT)])
    plsc.subcore_barrier()
    base = pl.multiple_of(wid * ETP, 8)

    def step(i, carry):
        off = pl.multiple_of(base + i * K, 8)
        pltpu.sync_copy(src_hbm.at[pl.ds(off, K)], six0)
        pltpu.sync_copy(dst_hbm.at[pl.ds(off, K)], dix0)
        pltpu.async_copy(u_hbm.at[six0], rows0, sem_g0).wait()
        pltpu.sync_copy(rows0, acc_sh.at[dix0], add=True)
        return carry

    lax.fori_loop(0, STEPS, step, 0)
    plsc.subcore_barrier()
    pltpu.sync_copy(acc_sh.at[pl.ds(pl.multiple_of(sid * RPT, 8), RPT)],
                    out_hbm.at[pl.ds(pl.multiple_of(cid * NP + sid * RPT, 8), RPT)])


_scat_call = pl.kernel(
    _scat_body,
    out_type=jax.ShapeDtypeStruct((2 * NP, HP), jnp.float32),
    mesh=_mesh,
    scratch_types=[
        pltpu.VMEM((K,), jnp.int32),
        pltpu.VMEM((K,), jnp.int32),
        pltpu.VMEM((K,), jnp.int32),
        pltpu.VMEM((K,), jnp.int32),
        pltpu.VMEM((K, HP), jnp.float32),
        pltpu.VMEM((K, HP), jnp.float32),
        pltpu.VMEM_SHARED((NP, HP), jnp.float32),
        pltpu.SemaphoreType.DMA,
        pltpu.SemaphoreType.DMA,
    ],
)


def _tc_prep_body(degp_ref, x_ref, w1_ref, u1_ref, d_ref):
    deg = jnp.sum(degp_ref[...], axis=1, keepdims=True) + 1.0
    d = lax.rsqrt(deg)
    h = jnp.dot(x_ref[...], w1_ref[...], preferred_element_type=jnp.float32)
    u1_ref[:, 0:H] = h * d
    u1_ref[:, H:HP] = jnp.zeros((N, HP - H), jnp.float32)
    d_ref[...] = d


def _tc_mid_body(sp_ref, u_ref, d_ref, b_ref, w2_ref, u2_ref):
    d = d_ref[...]
    s = (sp_ref[0:N, 0:H] + sp_ref[NP:NP + N, 0:H] + u_ref[0:N, 0:H])
    h = jnp.maximum(d * s + b_ref[...], 0.0)
    u2_ref[:, 0:H] = jnp.dot(h, w2_ref[...],
                             preferred_element_type=jnp.float32) * d
    u2_ref[:, H:HP] = jnp.zeros((N, HP - H), jnp.float32)


def _tc_final_body(sp_ref, u_ref, d_ref, b_ref, batch_ref, wl_ref, bl_ref,
                   out_ref):
    d = d_ref[...]
    s = (sp_ref[0:N, 0:H] + sp_ref[NP:NP + N, 0:H] + u_ref[0:N, 0:H])
    h = jnp.maximum(d * s + b_ref[...], 0.0)
    gids = lax.broadcasted_iota(jnp.int32, (1, G), 1)
    onehot = (batch_ref[...] == gids).astype(jnp.float32)
    sums = lax.dot_general(onehot, h, (((0,), (0,)), ((), ())),
                           preferred_element_type=jnp.float32)
    counts = jnp.sum(onehot, axis=0, keepdims=True)
    pooled = sums / jnp.maximum(counts, 1.0).reshape(G, 1)
    out_ref[...] = jnp.dot(pooled, wl_ref[...],
                           preferred_element_type=jnp.float32) + bl_ref[...]


def _tc_call(body, out_shape, n_in):
    return pl.pallas_call(
        body,
        out_shape=out_shape,
        in_specs=[pl.BlockSpec(memory_space=pltpu.VMEM)] * n_in,
        out_specs=(pl.BlockSpec(memory_space=pltpu.VMEM)
                   if not isinstance(out_shape, (list, tuple))
                   else [pl.BlockSpec(memory_space=pltpu.VMEM)] * len(out_shape)),
    )


_prep = _tc_call(_tc_prep_body,
                 [jax.ShapeDtypeStruct((N, HP), jnp.float32),
                  jax.ShapeDtypeStruct((N, 1), jnp.float32)], 3)
_mid = _tc_call(_tc_mid_body, jax.ShapeDtypeStruct((N, HP), jnp.float32), 5)
_final = _tc_call(_tc_final_body, jax.ShapeDtypeStruct((G, C), jnp.float32), 7)


@jax.jit
def kernel(x, edge_index, batch, W1, b1, W2, b2, Wl, bl):
    src = edge_index[0].astype(jnp.int32)
    dst = edge_index[1].astype(jnp.int32)
    npad = EP - E
    pad_iota = jnp.arange(npad, dtype=jnp.int32)
    src_p = jnp.concatenate([src, (pad_iota * 1237) % N])
    dst_p = jnp.concatenate([dst, N + pad_iota % (NP - N)])

    zeros_nh = jnp.zeros((NP, HP), jnp.float32)
    zeros_n = jnp.zeros((N,), jnp.float32)

    deg_parts = _deg_call(dst, zeros_n).reshape(NW, N).T
    u1, d = _prep(deg_parts, x, W1)
    s1 = _scat_call(u1, src_p, dst_p, zeros_nh)
    u2 = _mid(s1, u1, d, b1.reshape(1, H), W2)
    s2 = _scat_call(u2, src_p, dst_p, zeros_nh)
    return _final(s2, u2, d, b2.reshape(1, H),
                  batch.astype(jnp.int32).reshape(N, 1), Wl,
                  bl.reshape(1, C))


# speedup vs baseline: 2.5958x; 1.5645x over previous
import functools

import jax
import jax.numpy as jnp
from jax import lax
from jax.experimental import pallas as pl
from jax.experimental.pallas import tpu as pltpu
from jax.experimental.pallas import tpu_sc as plsc

N = 10000
E = 320000
F_IN = 128
H = 64
C = 10
G = 16

NC = 2
NS = 16
NW = NC * NS
ET = E // NW
K = 80
EP = 327680
ETP = EP // NW
STEPS = ETP // K
NP = 10240
RPT = NP // NS

_mesh = plsc.VectorSubcoreMesh(core_axis_name="c", subcore_axis_name="s")


def _deg_body(dst_hbm, zeros_hbm, out_hbm, idx_v, hist_v):
    cid = lax.axis_index("c")
    sid = lax.axis_index("s")
    wid = cid * NS + sid
    pltpu.sync_copy(zeros_hbm, hist_v)
    pltpu.sync_copy(dst_hbm.at[pl.ds(pl.multiple_of(wid * ET, 8), ET)], idx_v)
    ones16 = jnp.ones((16,), jnp.float32)

    def step(i, carry):
        idx16 = idx_v[pl.ds(i * 16, 16)]
        plsc.addupdate_scatter(hist_v, [idx16], ones16)
        return carry

    lax.fori_loop(0, ET // 16, step, 0)
    pltpu.sync_copy(hist_v, out_hbm.at[pl.ds(pl.multiple_of(wid * N, 8), N)])


_deg_call = pl.kernel(
    _deg_body,
    out_type=jax.ShapeDtypeStruct((NW * N,), jnp.float32),
    mesh=_mesh,
    scratch_types=[
        pltpu.VMEM((ET,), jnp.int32),
        pltpu.VMEM((N,), jnp.float32),
    ],
    compiler_params=pltpu.CompilerParams(needs_layout_passes=False),
)


HP = 128


def _scat_body(u_hbm, src_hbm, dst_hbm, zeros_hbm, out_hbm,
               six0, six1, dix0, dix1, rows0, rows1, acc_sh,
               sem_g0, sem_g1):
    cid = lax.axis_index("c")
    sid = lax.axis_index("s")
    wid = cid * NS + sid
    pltpu.sync_copy(zeros_hbm.at[pl.ds(pl.multiple_of(sid * RPT, 8), RPT)],
                    acc_sh.at[pl.ds(pl.multiple_of(sid * RPT, 8), RPT)])
    plsc.subcore_barrier()

    six = (six0, six1)
    dix = (dix0, dix1)
    rows = (rows0, rows1)
    sem_g = (sem_g0, sem_g1)
    base = pl.multiple_of(wid * ETP, 8)

    pltpu.sync_copy(src_hbm.at[pl.ds(base, K)], six0)
    pltpu.sync_copy(dst_hbm.at[pl.ds(base, K)], dix0)
    pltpu.async_copy(u_hbm.at[six0], rows0, sem_g0)

    def pair(q, carry):
        for b in range(2):
            i = q * 2 + b
            ob = 1 - b

            @pl.when(i + 1 < STEPS)
            def _():
                off = pl.multiple_of(base + (i + 1) * K, 8)
                pltpu.sync_copy(src_hbm.at[pl.ds(off, K)], six[ob])
                pltpu.sync_copy(dst_hbm.at[pl.ds(off, K)], dix[ob])

            pltpu.make_async_copy(u_hbm.at[pl.ds(0, K)], rows[b],
                                  sem_g[b]).wait()

            @pl.when(i + 1 < STEPS)
            def _():
                pltpu.async_copy(u_hbm.at[six[ob]], rows[ob], sem_g[ob])

            pltpu.sync_copy(rows[b], acc_sh.at[dix[b]], add=True)
        return carry

    lax.fori_loop(0, STEPS // 2, pair, 0)
    plsc.subcore_barrier()
    pltpu.sync_copy(acc_sh.at[pl.ds(pl.multiple_of(sid * RPT, 8), RPT)],
                    out_hbm.at[pl.ds(pl.multiple_of(cid * NP + sid * RPT, 8), RPT)])


_scat_call = pl.kernel(
    _scat_body,
    out_type=jax.ShapeDtypeStruct((2 * NP, HP), jnp.float32),
    mesh=_mesh,
    scratch_types=[
        pltpu.VMEM((K,), jnp.int32),
        pltpu.VMEM((K,), jnp.int32),
        pltpu.VMEM((K,), jnp.int32),
        pltpu.VMEM((K,), jnp.int32),
        pltpu.VMEM((K, HP), jnp.float32),
        pltpu.VMEM((K, HP), jnp.float32),
        pltpu.VMEM_SHARED((NP, HP), jnp.float32),
        pltpu.SemaphoreType.DMA,
        pltpu.SemaphoreType.DMA,
    ],
)


def _tc_prep_body(degp_ref, x_ref, w1_ref, u1_ref, d_ref):
    deg = jnp.sum(degp_ref[...], axis=1, keepdims=True) + 1.0
    d = lax.rsqrt(deg)
    h = jnp.dot(x_ref[...], w1_ref[...], preferred_element_type=jnp.float32)
    u1_ref[:, 0:H] = h * d
    u1_ref[:, H:HP] = jnp.zeros((N, HP - H), jnp.float32)
    d_ref[...] = d


def _tc_mid_body(sp_ref, u_ref, d_ref, b_ref, w2_ref, u2_ref):
    d = d_ref[...]
    s = (sp_ref[0:N, 0:H] + sp_ref[NP:NP + N, 0:H] + u_ref[0:N, 0:H])
    h = jnp.maximum(d * s + b_ref[...], 0.0)
    u2_ref[:, 0:H] = jnp.dot(h, w2_ref[...],
                             preferred_element_type=jnp.float32) * d
    u2_ref[:, H:HP] = jnp.zeros((N, HP - H), jnp.float32)


def _tc_final_body(sp_ref, u_ref, d_ref, b_ref, batch_ref, wl_ref, bl_ref,
                   out_ref):
    d = d_ref[...]
    s = (sp_ref[0:N, 0:H] + sp_ref[NP:NP + N, 0:H] + u_ref[0:N, 0:H])
    h = jnp.maximum(d * s + b_ref[...], 0.0)
    gids = lax.broadcasted_iota(jnp.int32, (1, G), 1)
    onehot = (batch_ref[...] == gids).astype(jnp.float32)
    sums = lax.dot_general(onehot, h, (((0,), (0,)), ((), ())),
                           preferred_element_type=jnp.float32)
    counts = jnp.sum(onehot, axis=0, keepdims=True)
    pooled = sums / jnp.maximum(counts, 1.0).reshape(G, 1)
    out_ref[...] = jnp.dot(pooled, wl_ref[...],
                           preferred_element_type=jnp.float32) + bl_ref[...]


def _tc_call(body, out_shape, n_in):
    return pl.pallas_call(
        body,
        out_shape=out_shape,
        in_specs=[pl.BlockSpec(memory_space=pltpu.VMEM)] * n_in,
        out_specs=(pl.BlockSpec(memory_space=pltpu.VMEM)
                   if not isinstance(out_shape, (list, tuple))
                   else [pl.BlockSpec(memory_space=pltpu.VMEM)] * len(out_shape)),
    )


_prep = _tc_call(_tc_prep_body,
                 [jax.ShapeDtypeStruct((N, HP), jnp.float32),
                  jax.ShapeDtypeStruct((N, 1), jnp.float32)], 3)
_mid = _tc_call(_tc_mid_body, jax.ShapeDtypeStruct((N, HP), jnp.float32), 5)
_final = _tc_call(_tc_final_body, jax.ShapeDtypeStruct((G, C), jnp.float32), 7)


@jax.jit
def kernel(x, edge_index, batch, W1, b1, W2, b2, Wl, bl):
    src = edge_index[0].astype(jnp.int32)
    dst = edge_index[1].astype(jnp.int32)
    npad = EP - E
    pad_iota = jnp.arange(npad, dtype=jnp.int32)
    src_p = jnp.concatenate([src, (pad_iota * 1237) % N])
    dst_p = jnp.concatenate([dst, N + pad_iota % (NP - N)])

    zeros_nh = jnp.zeros((NP, HP), jnp.float32)
    zeros_n = jnp.zeros((N,), jnp.float32)

    deg_parts = _deg_call(dst, zeros_n).reshape(NW, N).T
    u1, d = _prep(deg_parts, x, W1)
    s1 = _scat_call(u1, src_p, dst_p, zeros_nh)
    u2 = _mid(s1, u1, d, b1.reshape(1, H), W2)
    s2 = _scat_call(u2, src_p, dst_p, zeros_nh)
    return _final(s2, u2, d, b2.reshape(1, H),
                  batch.astype(jnp.int32).reshape(N, 1), Wl,
                  bl.reshape(1, C))


# speedup vs baseline: 2.9636x; 1.1417x over previous
import functools

import jax
import jax.numpy as jnp
from jax import lax
from jax.experimental import pallas as pl
from jax.experimental.pallas import tpu as pltpu
from jax.experimental.pallas import tpu_sc as plsc

N = 10000
E = 320000
F_IN = 128
H = 64
C = 10
G = 16

NC = 2
NS = 16
NW = NC * NS
ET = E // NW
K = 80
EP = 327680
ETP = EP // NW
STEPS = ETP // K
NP = 10240
RPT = NP // NS

_mesh = plsc.VectorSubcoreMesh(core_axis_name="c", subcore_axis_name="s")


def _deg_body(dst_hbm, zeros_hbm, out_hbm, idx_v, hist_v):
    cid = lax.axis_index("c")
    sid = lax.axis_index("s")
    wid = cid * NS + sid
    pltpu.sync_copy(zeros_hbm, hist_v)
    pltpu.sync_copy(dst_hbm.at[pl.ds(pl.multiple_of(wid * ET, 8), ET)], idx_v)
    ones16 = jnp.ones((16,), jnp.float32)

    def step(i, carry):
        idx16 = idx_v[pl.ds(i * 16, 16)]
        plsc.addupdate_scatter(hist_v, [idx16], ones16)
        return carry

    lax.fori_loop(0, ET // 16, step, 0)
    pltpu.sync_copy(hist_v, out_hbm.at[pl.ds(pl.multiple_of(wid * N, 8), N)])


_deg_call = pl.kernel(
    _deg_body,
    out_type=jax.ShapeDtypeStruct((NW * N,), jnp.float32),
    mesh=_mesh,
    scratch_types=[
        pltpu.VMEM((ET,), jnp.int32),
        pltpu.VMEM((N,), jnp.float32),
    ],
    compiler_params=pltpu.CompilerParams(needs_layout_passes=False),
)


HP = 128


def _scat_body(u_hbm, src_hbm, dst_hbm, zeros_hbm, out_hbm,
               six0, six1, dix0, dix1, rows0, rows1, acc_sh,
               sem_g0, sem_g1, sem_i0, sem_i1):
    cid = lax.axis_index("c")
    sid = lax.axis_index("s")
    wid = cid * NS + sid
    pltpu.sync_copy(zeros_hbm.at[pl.ds(pl.multiple_of(sid * RPT, 8), RPT)],
                    acc_sh.at[pl.ds(pl.multiple_of(sid * RPT, 8), RPT)])
    plsc.subcore_barrier()

    six = (six0, six1)
    dix = (dix0, dix1)
    rows = (rows0, rows1)
    sem_g = (sem_g0, sem_g1)
    sem_i = (sem_i0, sem_i1)
    base = pl.multiple_of(wid * ETP, 8)

    pltpu.sync_copy(src_hbm.at[pl.ds(base, K)], six0)
    pltpu.sync_copy(dst_hbm.at[pl.ds(base, K)], dix0)
    pltpu.async_copy(u_hbm.at[six0], rows0, sem_g0)
    pltpu.async_copy(src_hbm.at[pl.ds(pl.multiple_of(base + K, 8), K)],
                     six1, sem_i1)
    pltpu.async_copy(dst_hbm.at[pl.ds(pl.multiple_of(base + K, 8), K)],
                     dix1, sem_i1)

    def pair(q, carry):
        for b in range(2):
            i = q * 2 + b
            ob = 1 - b

            @pl.when(i + 1 < STEPS)
            def _():
                pltpu.make_async_copy(src_hbm.at[pl.ds(0, K)], six[ob],
                                      sem_i[ob]).wait()
                pltpu.make_async_copy(dst_hbm.at[pl.ds(0, K)], dix[ob],
                                      sem_i[ob]).wait()

            pltpu.make_async_copy(u_hbm.at[pl.ds(0, K)], rows[b],
                                  sem_g[b]).wait()

            @pl.when(i + 1 < STEPS)
            def _():
                pltpu.async_copy(u_hbm.at[six[ob]], rows[ob], sem_g[ob])

            pltpu.sync_copy(rows[b], acc_sh.at[dix[b]], add=True)

            @pl.when(i + 2 < STEPS)
            def _():
                off = pl.multiple_of(base + (i + 2) * K, 8)
                pltpu.async_copy(src_hbm.at[pl.ds(off, K)], six[b], sem_i[b])
                pltpu.async_copy(dst_hbm.at[pl.ds(off, K)], dix[b], sem_i[b])
        return carry

    lax.fori_loop(0, STEPS // 2, pair, 0)
    plsc.subcore_barrier()
    pltpu.sync_copy(acc_sh.at[pl.ds(pl.multiple_of(sid * RPT, 8), RPT)],
                    out_hbm.at[pl.ds(pl.multiple_of(cid * NP + sid * RPT, 8), RPT)])


_scat_call = pl.kernel(
    _scat_body,
    out_type=jax.ShapeDtypeStruct((2 * NP, HP), jnp.float32),
    mesh=_mesh,
    scratch_types=[
        pltpu.VMEM((K,), jnp.int32),
        pltpu.VMEM((K,), jnp.int32),
        pltpu.VMEM((K,), jnp.int32),
        pltpu.VMEM((K,), jnp.int32),
        pltpu.VMEM((K, HP), jnp.float32),
        pltpu.VMEM((K, HP), jnp.float32),
        pltpu.VMEM_SHARED((NP, HP), jnp.float32),
        pltpu.SemaphoreType.DMA,
        pltpu.SemaphoreType.DMA,
        pltpu.SemaphoreType.DMA,
        pltpu.SemaphoreType.DMA,
    ],
)


def _tc_prep_body(degp_ref, x_ref, w1_ref, u1_ref, d_ref):
    deg = jnp.sum(degp_ref[...], axis=1, keepdims=True) + 1.0
    d = lax.rsqrt(deg)
    h = jnp.dot(x_ref[...], w1_ref[...], preferred_element_type=jnp.float32)
    u1_ref[:, 0:H] = h * d
    u1_ref[:, H:HP] = jnp.zeros((N, HP - H), jnp.float32)
    d_ref[...] = d


def _tc_mid_body(sp_ref, u_ref, d_ref, b_ref, w2_ref, u2_ref):
    d = d_ref[...]
    s = (sp_ref[0:N, 0:H] + sp_ref[NP:NP + N, 0:H] + u_ref[0:N, 0:H])
    h = jnp.maximum(d * s + b_ref[...], 0.0)
    u2_ref[:, 0:H] = jnp.dot(h, w2_ref[...],
                             preferred_element_type=jnp.float32) * d
    u2_ref[:, H:HP] = jnp.zeros((N, HP - H), jnp.float32)


def _tc_final_body(sp_ref, u_ref, d_ref, b_ref, batch_ref, wl_ref, bl_ref,
                   out_ref):
    d = d_ref[...]
    s = (sp_ref[0:N, 0:H] + sp_ref[NP:NP + N, 0:H] + u_ref[0:N, 0:H])
    h = jnp.maximum(d * s + b_ref[...], 0.0)
    gids = lax.broadcasted_iota(jnp.int32, (1, G), 1)
    onehot = (batch_ref[...] == gids).astype(jnp.float32)
    sums = lax.dot_general(onehot, h, (((0,), (0,)), ((), ())),
                           preferred_element_type=jnp.float32)
    counts = jnp.sum(onehot, axis=0, keepdims=True)
    pooled = sums / jnp.maximum(counts, 1.0).reshape(G, 1)
    out_ref[...] = jnp.dot(pooled, wl_ref[...],
                           preferred_element_type=jnp.float32) + bl_ref[...]


def _tc_call(body, out_shape, n_in):
    return pl.pallas_call(
        body,
        out_shape=out_shape,
        in_specs=[pl.BlockSpec(memory_space=pltpu.VMEM)] * n_in,
        out_specs=(pl.BlockSpec(memory_space=pltpu.VMEM)
                   if not isinstance(out_shape, (list, tuple))
                   else [pl.BlockSpec(memory_space=pltpu.VMEM)] * len(out_shape)),
    )


_prep = _tc_call(_tc_prep_body,
                 [jax.ShapeDtypeStruct((N, HP), jnp.float32),
                  jax.ShapeDtypeStruct((N, 1), jnp.float32)], 3)
_mid = _tc_call(_tc_mid_body, jax.ShapeDtypeStruct((N, HP), jnp.float32), 5)
_final = _tc_call(_tc_final_body, jax.ShapeDtypeStruct((G, C), jnp.float32), 7)


@jax.jit
def kernel(x, edge_index, batch, W1, b1, W2, b2, Wl, bl):
    src = edge_index[0].astype(jnp.int32)
    dst = edge_index[1].astype(jnp.int32)
    npad = EP - E
    pad_iota = jnp.arange(npad, dtype=jnp.int32)
    src_p = jnp.concatenate([src, (pad_iota * 1237) % N])
    dst_p = jnp.concatenate([dst, N + pad_iota % (NP - N)])

    zeros_nh = jnp.zeros((NP, HP), jnp.float32)
    zeros_n = jnp.zeros((N,), jnp.float32)

    deg_parts = _deg_call(dst, zeros_n).reshape(NW, N).T
    u1, d = _prep(deg_parts, x, W1)
    s1 = _scat_call(u1, src_p, dst_p, zeros_nh)
    u2 = _mid(s1, u1, d, b1.reshape(1, H), W2)
    s2 = _scat_call(u2, src_p, dst_p, zeros_nh)
    return _final(s2, u2, d, b2.reshape(1, H),
                  batch.astype(jnp.int32).reshape(N, 1), Wl,
                  bl.reshape(1, C))


# speedup vs baseline: 3.3017x; 1.1141x over previous
import functools

import jax
import jax.numpy as jnp
from jax import lax
from jax.experimental import pallas as pl
from jax.experimental.pallas import tpu as pltpu
from jax.experimental.pallas import tpu_sc as plsc

N = 10000
E = 320000
F_IN = 128
H = 64
C = 10
G = 16

NC = 2
NS = 16
NW = NC * NS
ET = E // NW
K = 112
EP = 329728
ETP = EP // NW
STEPS = ETP // K
NP = 10240
RPT = NP // NS

_mesh = plsc.VectorSubcoreMesh(core_axis_name="c", subcore_axis_name="s")


def _deg_body(dst_hbm, zeros_hbm, out_hbm, idx_v, hist_v):
    cid = lax.axis_index("c")
    sid = lax.axis_index("s")
    wid = cid * NS + sid
    pltpu.sync_copy(zeros_hbm, hist_v)
    pltpu.sync_copy(dst_hbm.at[pl.ds(pl.multiple_of(wid * ET, 8), ET)], idx_v)
    ones16 = jnp.ones((16,), jnp.float32)

    def step(i, carry):
        idx16 = idx_v[pl.ds(i * 16, 16)]
        plsc.addupdate_scatter(hist_v, [idx16], ones16)
        return carry

    lax.fori_loop(0, ET // 16, step, 0)
    pltpu.sync_copy(hist_v, out_hbm.at[pl.ds(pl.multiple_of(wid * N, 8), N)])


_deg_call = pl.kernel(
    _deg_body,
    out_type=jax.ShapeDtypeStruct((NW * N,), jnp.float32),
    mesh=_mesh,
    scratch_types=[
        pltpu.VMEM((ET,), jnp.int32),
        pltpu.VMEM((N,), jnp.float32),
    ],
    compiler_params=pltpu.CompilerParams(needs_layout_passes=False),
)


HP = 128


def _scat_body(u_hbm, src_hbm, dst_hbm, zeros_hbm, out_hbm,
               six0, six1, dix0, dix1, rows0, rows1, acc_sh,
               sem_g0, sem_g1, sem_i0, sem_i1):
    cid = lax.axis_index("c")
    sid = lax.axis_index("s")
    wid = cid * NS + sid
    pltpu.sync_copy(zeros_hbm.at[pl.ds(pl.multiple_of(sid * RPT, 8), RPT)],
                    acc_sh.at[pl.ds(pl.multiple_of(sid * RPT, 8), RPT)])
    plsc.subcore_barrier()

    six = (six0, six1)
    dix = (dix0, dix1)
    rows = (rows0, rows1)
    sem_g = (sem_g0, sem_g1)
    sem_i = (sem_i0, sem_i1)
    base = pl.multiple_of(wid * ETP, 8)

    pltpu.sync_copy(src_hbm.at[pl.ds(base, K)], six0)
    pltpu.sync_copy(dst_hbm.at[pl.ds(base, K)], dix0)
    pltpu.async_copy(u_hbm.at[six0], rows0, sem_g0)
    pltpu.async_copy(src_hbm.at[pl.ds(pl.multiple_of(base + K, 8), K)],
                     six1, sem_i1)
    pltpu.async_copy(dst_hbm.at[pl.ds(pl.multiple_of(base + K, 8), K)],
                     dix1, sem_i1)

    def pair(q, carry):
        for b in range(2):
            i = q * 2 + b
            ob = 1 - b

            @pl.when(i + 1 < STEPS)
            def _():
                pltpu.make_async_copy(src_hbm.at[pl.ds(0, K)], six[ob],
                                      sem_i[ob]).wait()
                pltpu.make_async_copy(dst_hbm.at[pl.ds(0, K)], dix[ob],
                                      sem_i[ob]).wait()

            pltpu.make_async_copy(u_hbm.at[pl.ds(0, K)], rows[b],
                                  sem_g[b]).wait()

            @pl.when(i + 1 < STEPS)
            def _():
                pltpu.async_copy(u_hbm.at[six[ob]], rows[ob], sem_g[ob])

            pltpu.sync_copy(rows[b], acc_sh.at[dix[b]], add=True)

            @pl.when(i + 2 < STEPS)
            def _():
                off = pl.multiple_of(base + (i + 2) * K, 8)
                pltpu.async_copy(src_hbm.at[pl.ds(off, K)], six[b], sem_i[b])
                pltpu.async_copy(dst_hbm.at[pl.ds(off, K)], dix[b], sem_i[b])
        return carry

    lax.fori_loop(0, STEPS // 2, pair, 0)
    plsc.subcore_barrier()
    pltpu.sync_copy(acc_sh.at[pl.ds(pl.multiple_of(sid * RPT, 8), RPT)],
                    out_hbm.at[pl.ds(pl.multiple_of(cid * NP + sid * RPT, 8), RPT)])


_scat_call = pl.kernel(
    _scat_body,
    out_type=jax.ShapeDtypeStruct((2 * NP, HP), jnp.float32),
    mesh=_mesh,
    scratch_types=[
        pltpu.VMEM((K,), jnp.int32),
        pltpu.VMEM((K,), jnp.int32),
        pltpu.VMEM((K,), jnp.int32),
        pltpu.VMEM((K,), jnp.int32),
        pltpu.VMEM((K, HP), jnp.float32),
        pltpu.VMEM((K, HP), jnp.float32),
        pltpu.VMEM_SHARED((NP, HP), jnp.float32),
        pltpu.SemaphoreType.DMA,
        pltpu.SemaphoreType.DMA,
        pltpu.SemaphoreType.DMA,
        pltpu.SemaphoreType.DMA,
    ],
)


def _tc_prep_body(degp_ref, x_ref, w1_ref, u1_ref, d_ref):
    deg = jnp.sum(degp_ref[...], axis=1, keepdims=True) + 1.0
    d = lax.rsqrt(deg)
    h = jnp.dot(x_ref[...], w1_ref[...], preferred_element_type=jnp.float32)
    u1_ref[:, 0:H] = h * d
    u1_ref[:, H:HP] = jnp.zeros((N, HP - H), jnp.float32)
    d_ref[...] = d


def _tc_mid_body(sp_ref, u_ref, d_ref, b_ref, w2_ref, u2_ref):
    d = d_ref[...]
    s = (sp_ref[0:N, 0:H] + sp_ref[NP:NP + N, 0:H] + u_ref[0:N, 0:H])
    h = jnp.maximum(d * s + b_ref[...], 0.0)
    u2_ref[:, 0:H] = jnp.dot(h, w2_ref[...],
                             preferred_element_type=jnp.float32) * d
    u2_ref[:, H:HP] = jnp.zeros((N, HP - H), jnp.float32)


def _tc_final_body(sp_ref, u_ref, d_ref, b_ref, batch_ref, wl_ref, bl_ref,
                   out_ref):
    d = d_ref[...]
    s = (sp_ref[0:N, 0:H] + sp_ref[NP:NP + N, 0:H] + u_ref[0:N, 0:H])
    h = jnp.maximum(d * s + b_ref[...], 0.0)
    gids = lax.broadcasted_iota(jnp.int32, (1, G), 1)
    onehot = (batch_ref[...] == gids).astype(jnp.float32)
    sums = lax.dot_general(onehot, h, (((0,), (0,)), ((), ())),
                           preferred_element_type=jnp.float32)
    counts = jnp.sum(onehot, axis=0, keepdims=True)
    pooled = sums / jnp.maximum(counts, 1.0).reshape(G, 1)
    out_ref[...] = jnp.dot(pooled, wl_ref[...],
                           preferred_element_type=jnp.float32) + bl_ref[...]


def _tc_call(body, out_shape, n_in):
    return pl.pallas_call(
        body,
        out_shape=out_shape,
        in_specs=[pl.BlockSpec(memory_space=pltpu.VMEM)] * n_in,
        out_specs=(pl.BlockSpec(memory_space=pltpu.VMEM)
                   if not isinstance(out_shape, (list, tuple))
                   else [pl.BlockSpec(memory_space=pltpu.VMEM)] * len(out_shape)),
    )


_prep = _tc_call(_tc_prep_body,
                 [jax.ShapeDtypeStruct((N, HP), jnp.float32),
                  jax.ShapeDtypeStruct((N, 1), jnp.float32)], 3)
_mid = _tc_call(_tc_mid_body, jax.ShapeDtypeStruct((N, HP), jnp.float32), 5)
_final = _tc_call(_tc_final_body, jax.ShapeDtypeStruct((G, C), jnp.float32), 7)


@jax.jit
def kernel(x, edge_index, batch, W1, b1, W2, b2, Wl, bl):
    src = edge_index[0].astype(jnp.int32)
    dst = edge_index[1].astype(jnp.int32)
    npad = EP - E
    pad_iota = jnp.arange(npad, dtype=jnp.int32)
    src_p = jnp.concatenate([src, (pad_iota * 1237) % N])
    dst_p = jnp.concatenate([dst, N + pad_iota % (NP - N)])

    zeros_nh = jnp.zeros((NP, HP), jnp.float32)
    zeros_n = jnp.zeros((N,), jnp.float32)

    deg_parts = _deg_call(dst, zeros_n).reshape(NW, N).T
    u1, d = _prep(deg_parts, x, W1)
    s1 = _scat_call(u1, src_p, dst_p, zeros_nh)
    u2 = _mid(s1, u1, d, b1.reshape(1, H), W2)
    s2 = _scat_call(u2, src_p, dst_p, zeros_nh)
    return _final(s2, u2, d, b2.reshape(1, H),
                  batch.astype(jnp.int32).reshape(N, 1), Wl,
                  bl.reshape(1, C))


# speedup vs baseline: 3.3072x; 1.0017x over previous
import jax
import jax.numpy as jnp
from jax import lax
from jax.experimental import pallas as pl
from jax.experimental.pallas import tpu as pltpu
from jax.experimental.pallas import tpu_sc as plsc

N = 10000
E = 320000
F_IN = 128
H = 64
C = 10
G = 16

NC = 2
NS = 16
NW = NC * NS
ET = E // NW
K = 112
EP = 329728
ETP = EP // NW
STEPS = ETP // K
NP = 10240
RPT = NP // NS

_mesh = plsc.VectorSubcoreMesh(core_axis_name="c", subcore_axis_name="s")


def _deg_body(dst_hbm, zeros_hbm, out_hbm, idx_v, hist_v):
    cid = lax.axis_index("c")
    sid = lax.axis_index("s")
    wid = cid * NS + sid
    pltpu.sync_copy(zeros_hbm, hist_v)
    pltpu.sync_copy(dst_hbm.at[pl.ds(pl.multiple_of(wid * ET, 8), ET)], idx_v)
    ones16 = jnp.ones((16,), jnp.float32)

    def step(i, carry):
        idx16 = idx_v[pl.ds(i * 16, 16)]
        plsc.addupdate_scatter(hist_v, [idx16], ones16)
        return carry

    lax.fori_loop(0, ET // 16, step, 0)
    pltpu.sync_copy(hist_v, out_hbm.at[pl.ds(pl.multiple_of(wid * N, 8), N)])


_deg_call = pl.kernel(
    _deg_body,
    out_type=jax.ShapeDtypeStruct((NW * N,), jnp.float32),
    mesh=_mesh,
    scratch_types=[
        pltpu.VMEM((ET,), jnp.int32),
        pltpu.VMEM((N,), jnp.float32),
    ],
    compiler_params=pltpu.CompilerParams(needs_layout_passes=False),
)


HP = 128


def _scat_body(u_hbm, src_hbm, dst_hbm, zeros_hbm, out_hbm,
               six0, six1, dix0, dix1, rows0, rows1, acc_sh,
               sem_g0, sem_g1, sem_i0, sem_i1):
    cid = lax.axis_index("c")
    sid = lax.axis_index("s")
    wid = cid * NS + sid
    pltpu.sync_copy(zeros_hbm.at[pl.ds(pl.multiple_of(sid * RPT, 8), RPT)],
                    acc_sh.at[pl.ds(pl.multiple_of(sid * RPT, 8), RPT)])
    plsc.subcore_barrier()

    six = (six0, six1)
    dix = (dix0, dix1)
    rows = (rows0, rows1)
    sem_g = (sem_g0, sem_g1)
    sem_i = (sem_i0, sem_i1)
    base = pl.multiple_of(wid * ETP, 8)

    pltpu.sync_copy(src_hbm.at[pl.ds(base, K)], six0)
    pltpu.sync_copy(dst_hbm.at[pl.ds(base, K)], dix0)
    pltpu.async_copy(u_hbm.at[six0], rows0, sem_g0)
    pltpu.async_copy(src_hbm.at[pl.ds(pl.multiple_of(base + K, 8), K)],
                     six1, sem_i1)
    pltpu.async_copy(dst_hbm.at[pl.ds(pl.multiple_of(base + K, 8), K)],
                     dix1, sem_i1)

    def pair(q, carry):
        for b in range(2):
            i = q * 2 + b
            ob = 1 - b

            @pl.when(i + 1 < STEPS)
            def _():
                pltpu.make_async_copy(src_hbm.at[pl.ds(0, K)], six[ob],
                                      sem_i[ob]).wait()
                pltpu.make_async_copy(dst_hbm.at[pl.ds(0, K)], dix[ob],
                                      sem_i[ob]).wait()

            pltpu.make_async_copy(u_hbm.at[pl.ds(0, K)], rows[b],
                                  sem_g[b]).wait()

            @pl.when(i + 1 < STEPS)
            def _():
                pltpu.async_copy(u_hbm.at[six[ob]], rows[ob], sem_g[ob])

            pltpu.sync_copy(rows[b], acc_sh.at[dix[b]], add=True)

            @pl.when(i + 2 < STEPS)
            def _():
                off = pl.multiple_of(base + (i + 2) * K, 8)
                pltpu.async_copy(src_hbm.at[pl.ds(off, K)], six[b], sem_i[b])
                pltpu.async_copy(dst_hbm.at[pl.ds(off, K)], dix[b], sem_i[b])
        return carry

    lax.fori_loop(0, STEPS // 2, pair, 0)
    plsc.subcore_barrier()
    pltpu.sync_copy(acc_sh.at[pl.ds(pl.multiple_of(sid * RPT, 8), RPT)],
                    out_hbm.at[pl.ds(pl.multiple_of(cid * NP + sid * RPT, 8), RPT)])


_scat_call = pl.kernel(
    _scat_body,
    out_type=jax.ShapeDtypeStruct((2 * NP, HP), jnp.float32),
    mesh=_mesh,
    scratch_types=[
        pltpu.VMEM((K,), jnp.int32),
        pltpu.VMEM((K,), jnp.int32),
        pltpu.VMEM((K,), jnp.int32),
        pltpu.VMEM((K,), jnp.int32),
        pltpu.VMEM((K, HP), jnp.float32),
        pltpu.VMEM((K, HP), jnp.float32),
        pltpu.VMEM_SHARED((NP, HP), jnp.float32),
        pltpu.SemaphoreType.DMA,
        pltpu.SemaphoreType.DMA,
        pltpu.SemaphoreType.DMA,
        pltpu.SemaphoreType.DMA,
    ],
)


def _tc_prep_body(degp_ref, x_ref, w1_ref, u1_ref, d_ref):
    deg = jnp.sum(degp_ref[...], axis=1, keepdims=True) + 1.0
    d = lax.rsqrt(deg)
    h = jnp.dot(x_ref[...], w1_ref[...], preferred_element_type=jnp.float32)
    u1_ref[:, 0:H] = h * d
    u1_ref[:, H:HP] = jnp.zeros((N, HP - H), jnp.float32)
    d_ref[...] = d


def _tc_mid_body(sp_ref, u_ref, d_ref, b_ref, w2_ref, u2_ref):
    d = d_ref[...]
    s = (sp_ref[0:N, 0:H] + sp_ref[NP:NP + N, 0:H] + u_ref[0:N, 0:H])
    h = jnp.maximum(d * s + b_ref[...], 0.0)
    u2_ref[:, 0:H] = jnp.dot(h, w2_ref[...],
                             preferred_element_type=jnp.float32) * d
    u2_ref[:, H:HP] = jnp.zeros((N, HP - H), jnp.float32)


def _tc_final_body(sp_ref, u_ref, d_ref, b_ref, batch_ref, wl_ref, bl_ref,
                   out_ref):
    d = d_ref[...]
    s = (sp_ref[0:N, 0:H] + sp_ref[NP:NP + N, 0:H] + u_ref[0:N, 0:H])
    h = jnp.maximum(d * s + b_ref[...], 0.0)
    gids = lax.broadcasted_iota(jnp.int32, (1, G), 1)
    onehot = (batch_ref[...] == gids).astype(jnp.float32)
    sums = lax.dot_general(onehot, h, (((0,), (0,)), ((), ())),
                           preferred_element_type=jnp.float32)
    counts = jnp.sum(onehot, axis=0, keepdims=True)
    pooled = sums / jnp.maximum(counts, 1.0).reshape(G, 1)
    out_ref[...] = jnp.dot(pooled, wl_ref[...],
                           preferred_element_type=jnp.float32) + bl_ref[...]


def _tc_call(body, out_shape, n_in):
    return pl.pallas_call(
        body,
        out_shape=out_shape,
        in_specs=[pl.BlockSpec(memory_space=pltpu.VMEM)] * n_in,
        out_specs=(pl.BlockSpec(memory_space=pltpu.VMEM)
                   if not isinstance(out_shape, (list, tuple))
                   else [pl.BlockSpec(memory_space=pltpu.VMEM)] * len(out_shape)),
    )


_prep = _tc_call(_tc_prep_body,
                 [jax.ShapeDtypeStruct((N, HP), jnp.float32),
                  jax.ShapeDtypeStruct((N, 1), jnp.float32)], 3)
_mid = _tc_call(_tc_mid_body, jax.ShapeDtypeStruct((N, HP), jnp.float32), 5)
_final = _tc_call(_tc_final_body, jax.ShapeDtypeStruct((G, C), jnp.float32), 7)


@jax.jit
def kernel(x, edge_index, batch, W1, b1, W2, b2, Wl, bl):
    src = edge_index[0].astype(jnp.int32)
    dst = edge_index[1].astype(jnp.int32)
    npad = EP - E
    pad_iota = jnp.arange(npad, dtype=jnp.int32)
    src_p = jnp.concatenate([src, (pad_iota * 1237) % N])
    dst_p = jnp.concatenate([dst, N + pad_iota % (NP - N)])

    zeros_nh = jnp.zeros((NP, HP), jnp.float32)
    zeros_n = jnp.zeros((N,), jnp.float32)

    deg_parts = _deg_call(dst, zeros_n).reshape(NW, N).T
    u1, d = _prep(deg_parts, x, W1)
    s1 = _scat_call(u1, src_p, dst_p, zeros_nh)
    u2 = _mid(s1, u1, d, b1.reshape(1, H), W2)
    s2 = _scat_call(u2, src_p, dst_p, zeros_nh)
    return _final(s2, u2, d, b2.reshape(1, H),
                  batch.astype(jnp.int32).reshape(N, 1), Wl,
                  bl.reshape(1, C))
